# baseline (speedup 1.0000x reference)
import functools

import jax
import jax.numpy as jnp
from jax import lax
from jax.experimental import pallas as pl
from jax.experimental.pallas import tpu as pltpu
from jax.experimental.pallas import tpu_sc as plsc

_RES = 32
_EPS = 1e-06
_NVOX = _RES * _RES * _RES
_L = 16
_NC = 2
_NS = 16
_NW = _NC * _NS
_CH = 2048
_D = 8
_HALF = _NVOX // 2


def _norm_body(c_ref, nc_ref, idx_ref):
    c = c_ref[0]
    N = c.shape[1]
    mean = jnp.mean(c, axis=1, keepdims=True)
    cen = c - mean
    norms = jnp.sqrt(jnp.sum(cen * cen, axis=0, keepdims=True))
    red = jnp.max(norms)
    nc = cen / (red * 2.0 + _EPS) + 0.5
    nc = jnp.clip(nc * float(_RES), 0.0, float(_RES - 1))
    nc_ref[0] = nc
    vox = jnp.round(nc).astype(jnp.int32)
    flat = vox[0] * (_RES * _RES) + vox[1] * _RES + vox[2]
    packed = flat[: N // 2] | (flat[N // 2:] << 16)
    idx_ref[0] = packed[None]


def _normalize(coords):
    B, _, N = coords.shape
    return pl.pallas_call(
        _norm_body,
        grid=(B,),
        in_specs=[pl.BlockSpec((1, 3, N), lambda b: (b, 0, 0))],
        out_specs=[
            pl.BlockSpec((1, 3, N), lambda b: (b, 0, 0)),
            pl.BlockSpec((1, 1, N // 2), lambda b: (b, 0, 0)),
        ],
        out_shape=[
            jax.ShapeDtypeStruct((B, 3, N), jnp.float32),
            jax.ShapeDtypeStruct((B, 1, N // 2), jnp.int32),
        ],
    )(coords)


def _make_main_kernel(B, C, N):
    mesh = plsc.VectorSubcoreMesh(core_axis_name="c", subcore_axis_name="s")
    cpt = (B * C) // _NW
    tpb = _NW // B
    nh = N // 2
    nchunk = nh // _CH
    assert nchunk % _D == 0 and nchunk == 2 * _D

    @functools.partial(
        pl.kernel,
        mesh=mesh,
        out_type=jax.ShapeDtypeStruct((B, C, _NVOX), jnp.float32),
        compiler_params=pltpu.CompilerParams(needs_layout_passes=False),
        scratch_types=[
            pltpu.VMEM((_NVOX,), jnp.float32),
            pltpu.VMEM((_NVOX,), jnp.float32),
            [[pltpu.VMEM((_CH,), jnp.float32) for _ in range(2)]
             for _ in range(_D)],
            [pltpu.VMEM((_CH,), jnp.int32) for _ in range(_D)],
            [[pltpu.SemaphoreType.DMA for _ in range(2)] for _ in range(_D)],
            [pltpu.SemaphoreType.DMA for _ in range(_D)],
            [pltpu.SemaphoreType.DMA for _ in range(2)],
        ],
    )
    def main_k(feat_hbm, idx_hbm, out_hbm, acc_v, recip_v, fbufs, ibufs,
               fsems, isems, osems):
        wid = lax.axis_index("s") * _NC + lax.axis_index("c")
        b = wid // tpb
        g = wid % tpb
        ch0 = g * cpt

        def issue_idx(k, d):
            return pltpu.async_copy(
                idx_hbm.at[b, 0, pl.ds(k * _CH, _CH)], ibufs[d], isems[d])

        def issue_feat(ch, k, d):
            return [
                pltpu.async_copy(
                    feat_hbm.at[b, ch, pl.ds(h * nh + k * _CH, _CH)],
                    fbufs[d][h], fsems[d][h])
                for h in range(2)
            ]

        for d in range(_D):
            issue_idx(d, d)
            issue_feat(ch0, d, d)

        zeros = jnp.zeros((_L,), jnp.float32)
        ones = jnp.ones((_L,), jnp.float32)

        def zero_a(i, cc):
            for u in range(8):
                acc_v[pl.ds((i * 8 + u) * _L, _L)] = zeros
            return cc

        lax.fori_loop(0, _NVOX // (8 * _L), zero_a, 0)

        for k in range(nchunk):
            d = k % _D
            pltpu.make_async_copy(
                idx_hbm.at[b, 0, pl.ds(0, _CH)], ibufs[d], isems[d]).wait()

            def cnt_body(i, cc, ib=ibufs[d]):
                pks = [plsc.bitcast(ib[pl.ds((i * 4 + u) * _L, _L)], jnp.int16)
                       for u in range(4)]
                ups = [plsc.unpack(pk, format=plsc.PackFormat.INTERLEAVED)
                       for pk in pks]
                for lo, hi in ups:
                    plsc.addupdate_scatter(acc_v, [lo], ones)
                    plsc.addupdate_scatter(acc_v, [hi], ones)
                return cc

            lax.fori_loop(0, _CH // (4 * _L), cnt_body, 0)
            issue_idx(k + _D if k < _D else k - _D, d)

        def recip_body(i, cc):
            offs = [(i * 4 + u) * _L for u in range(4)]
            cs = [acc_v[pl.ds(off, _L)] for off in offs]
            for off, c16 in zip(offs, cs):
                recip_v[pl.ds(off, _L)] = ones / jnp.maximum(c16, ones)
            return cc

        lax.fori_loop(0, _NVOX // (4 * _L), recip_body, 0)

        def chan_body(ci, cc):
            ch = ch0 + ci
            for h in range(2):
                @pl.when(ci > 0)
                def _(h=h):
                    pltpu.make_async_copy(
                        acc_v.at[pl.ds(h * _HALF, _HALF)],
                        out_hbm.at[b, ch0, pl.ds(h * _HALF, _HALF)],
                        osems[h]).wait()

                def zb(i, cc2, h=h):
                    for u in range(8):
                        acc_v[pl.ds(h * _HALF + (i * 8 + u) * _L, _L)] = zeros
                    return cc2

                lax.fori_loop(0, _HALF // (8 * _L), zb, 0)

            for k in range(nchunk):
                d = k % _D
                for h in range(2):
                    pltpu.make_async_copy(
                        feat_hbm.at[b, ch0, pl.ds(h * nh, _CH)],
                        fbufs[d][h], fsems[d][h]).wait()
                pltpu.make_async_copy(
                    idx_hbm.at[b, 0, pl.ds(0, _CH)], ibufs[d], isems[d]).wait()
                bufA, bufB = fbufs[d]

                def sc_body(i, cc2, bufA=bufA, bufB=bufB, ib=ibufs[d]):
                    locs = [(i * 8 + u) * _L for u in range(8)]
                    pks = [plsc.bitcast(ib[pl.ds(loc, _L)], jnp.int16)
                           for loc in locs]
                    vas = [bufA[pl.ds(loc, _L)] for loc in locs]
                    vbs = [bufB[pl.ds(loc, _L)] for loc in locs]
                    ups = [plsc.unpack(pk, format=plsc.PackFormat.INTERLEAVED)
                           for pk in pks]
                    for u in range(8):
                        lo, hi = ups[u]
                        plsc.addupdate_scatter(acc_v, [lo], vas[u])
                        plsc.addupdate_scatter(acc_v, [hi], vbs[u])
                    return cc2

                lax.fori_loop(0, _CH // (8 * _L), sc_body, 0)
                if k < nchunk - _D:
                    issue_feat(ch, k + _D, d)
                    issue_idx(k + _D, d)
                else:
                    @pl.when(ci < cpt - 1)
                    def _(k=k, d=d):
                        issue_feat(ch + 1, k + _D - nchunk, d)
                    issue_idx(k + _D - nchunk, d)

            for h in range(2):
                def wb(i, cc2, h=h):
                    offs = [h * _HALF + (i * 4 + u) * _L for u in range(4)]
                    accs = [acc_v[pl.ds(off, _L)] for off in offs]
                    rs = [recip_v[pl.ds(off, _L)] for off in offs]
                    for off, a, r in zip(offs, accs, rs):
                        acc_v[pl.ds(off, _L)] = a * r
                    return cc2

                lax.fori_loop(0, _HALF // (4 * _L), wb, 0)
                pltpu.async_copy(
                    acc_v.at[pl.ds(h * _HALF, _HALF)],
                    out_hbm.at[b, ch, pl.ds(h * _HALF, _HALF)],
                    osems[h])
            return cc

        lax.fori_loop(0, cpt, chan_body, 0)

        for h in range(2):
            pltpu.make_async_copy(
                acc_v.at[pl.ds(h * _HALF, _HALF)],
                out_hbm.at[b, ch0, pl.ds(h * _HALF, _HALF)],
                osems[h]).wait()
        for d in range(_D):
            pltpu.make_async_copy(
                idx_hbm.at[b, 0, pl.ds(0, _CH)], ibufs[d], isems[d]).wait()

    return main_k


def kernel(features, coords):
    B, C, N = features.shape
    nc, idx_pack = _normalize(coords)
    avg = _make_main_kernel(B, C, N)(features, idx_pack)
    return avg.reshape(B, C, _RES, _RES, _RES), nc


# speedup vs baseline: 1.0034x; 1.0034x over previous
import functools

import jax
import jax.numpy as jnp
from jax import lax
from jax.experimental import pallas as pl
from jax.experimental.pallas import tpu as pltpu
from jax.experimental.pallas import tpu_sc as plsc

_RES = 32
_EPS = 1e-06
_NVOX = _RES * _RES * _RES
_L = 16
_NC = 2
_NS = 16
_NW = _NC * _NS
_CH = 2048
_D = 8
_HALF = _NVOX // 2


def _norm_body(c_ref, nc_ref, idx_ref):
    c = c_ref[0]
    N = c.shape[1]
    mean = jnp.mean(c, axis=1, keepdims=True)
    cen = c - mean
    norms = jnp.sqrt(jnp.sum(cen * cen, axis=0, keepdims=True))
    red = jnp.max(norms)
    nc = cen / (red * 2.0 + _EPS) + 0.5
    nc = jnp.clip(nc * float(_RES), 0.0, float(_RES - 1))
    nc_ref[0] = nc
    vox = jnp.round(nc).astype(jnp.int32)
    flat = vox[0] * (_RES * _RES) + vox[1] * _RES + vox[2]
    packed = flat[: N // 2] | (flat[N // 2:] << 16)
    idx_ref[0] = packed[None]


def _normalize(coords):
    B, _, N = coords.shape
    return pl.pallas_call(
        _norm_body,
        grid=(B,),
        in_specs=[pl.BlockSpec((1, 3, N), lambda b: (b, 0, 0))],
        out_specs=[
            pl.BlockSpec((1, 3, N), lambda b: (b, 0, 0)),
            pl.BlockSpec((1, 1, N // 2), lambda b: (b, 0, 0)),
        ],
        out_shape=[
            jax.ShapeDtypeStruct((B, 3, N), jnp.float32),
            jax.ShapeDtypeStruct((B, 1, N // 2), jnp.int32),
        ],
    )(coords)


def _make_main_kernel(B, C, N):
    mesh = plsc.VectorSubcoreMesh(core_axis_name="c", subcore_axis_name="s")
    cpt = (B * C) // _NW
    tpb = _NW // B
    nh = N // 2
    nchunk = nh // _CH
    assert nchunk % _D == 0 and nchunk == 2 * _D

    @functools.partial(
        pl.kernel,
        mesh=mesh,
        out_type=jax.ShapeDtypeStruct((B, C, _NVOX), jnp.float32),
        compiler_params=pltpu.CompilerParams(needs_layout_passes=False),
        scratch_types=[
            pltpu.VMEM((_NVOX,), jnp.float32),
            pltpu.VMEM((_NVOX,), jnp.float32),
            [[pltpu.VMEM((_CH,), jnp.float32) for _ in range(2)]
             for _ in range(_D)],
            [pltpu.VMEM((_CH,), jnp.int32) for _ in range(_D)],
            [[pltpu.SemaphoreType.DMA for _ in range(2)] for _ in range(_D)],
            [pltpu.SemaphoreType.DMA for _ in range(_D)],
            [pltpu.SemaphoreType.DMA for _ in range(2)],
        ],
    )
    def main_k(feat_hbm, idx_hbm, out_hbm, acc_v, recip_v, fbufs, ibufs,
               fsems, isems, osems):
        wid = lax.axis_index("s") * _NC + lax.axis_index("c")
        b = wid // tpb
        g = wid % tpb
        ch0 = g * cpt

        def issue_idx(k, d):
            return pltpu.async_copy(
                idx_hbm.at[b, 0, pl.ds(k * _CH, _CH)], ibufs[d], isems[d])

        def issue_feat(ch, k, d):
            return [
                pltpu.async_copy(
                    feat_hbm.at[b, ch, pl.ds(h * nh + k * _CH, _CH)],
                    fbufs[d][h], fsems[d][h])
                for h in range(2)
            ]

        for d in range(_D):
            issue_idx(d, d)
            issue_feat(ch0, d, d)

        zeros = jnp.zeros((_L,), jnp.float32)
        ones = jnp.ones((_L,), jnp.float32)

        def zero_a(i, cc):
            for u in range(8):
                acc_v[pl.ds((i * 8 + u) * _L, _L)] = zeros
            return cc

        lax.fori_loop(0, _NVOX // (8 * _L), zero_a, 0)

        for k in range(nchunk):
            d = k % _D
            pltpu.make_async_copy(
                idx_hbm.at[b, 0, pl.ds(0, _CH)], ibufs[d], isems[d]).wait()

            def cnt_body(i, cc, ib=ibufs[d]):
                pks = [plsc.bitcast(ib[pl.ds((i * 4 + u) * _L, _L)], jnp.int16)
                       for u in range(4)]
                ups = [plsc.unpack(pk, format=plsc.PackFormat.INTERLEAVED)
                       for pk in pks]
                for lo, hi in ups:
                    plsc.addupdate_scatter(acc_v, [lo], ones)
                    plsc.addupdate_scatter(acc_v, [hi], ones)
                return cc

            lax.fori_loop(0, _CH // (4 * _L), cnt_body, 0)
            issue_idx(k + _D if k < _D else k - _D, d)

        def recip_body(i, cc):
            offs = [(i * 4 + u) * _L for u in range(4)]
            cs = [acc_v[pl.ds(off, _L)] for off in offs]
            for off, c16 in zip(offs, cs):
                recip_v[pl.ds(off, _L)] = ones / jnp.maximum(c16, ones)
            return cc

        lax.fori_loop(0, _NVOX // (4 * _L), recip_body, 0)

        def chan_body(ci, cc):
            ch = ch0 + ci
            for h in range(2):
                @pl.when(ci > 0)
                def _(h=h):
                    pltpu.make_async_copy(
                        acc_v.at[pl.ds(h * _HALF, _HALF)],
                        out_hbm.at[b, ch0, pl.ds(h * _HALF, _HALF)],
                        osems[h]).wait()

                def zb(i, cc2, h=h):
                    for u in range(8):
                        acc_v[pl.ds(h * _HALF + (i * 8 + u) * _L, _L)] = zeros
                    return cc2

                lax.fori_loop(0, _HALF // (8 * _L), zb, 0)

            for k in range(nchunk):
                d = k % _D
                for h in range(2):
                    pltpu.make_async_copy(
                        feat_hbm.at[b, ch0, pl.ds(h * nh, _CH)],
                        fbufs[d][h], fsems[d][h]).wait()
                pltpu.make_async_copy(
                    idx_hbm.at[b, 0, pl.ds(0, _CH)], ibufs[d], isems[d]).wait()
                bufA, bufB = fbufs[d]

                def sc_body(i, cc2, bufA=bufA, bufB=bufB, ib=ibufs[d]):
                    locs = [(i * 4 + u) * _L for u in range(4)]
                    pks = [plsc.bitcast(ib[pl.ds(loc, _L)], jnp.int16)
                           for loc in locs]
                    vas = [bufA[pl.ds(loc, _L)] for loc in locs]
                    vbs = [bufB[pl.ds(loc, _L)] for loc in locs]
                    ups = [plsc.unpack(pk, format=plsc.PackFormat.INTERLEAVED)
                           for pk in pks]
                    for u in range(4):
                        lo, hi = ups[u]
                        plsc.addupdate_scatter(acc_v, [lo], vas[u])
                        plsc.addupdate_scatter(acc_v, [hi], vbs[u])
                    return cc2

                lax.fori_loop(0, _CH // (4 * _L), sc_body, 0)
                if k < nchunk - _D:
                    issue_feat(ch, k + _D, d)
                    issue_idx(k + _D, d)
                else:
                    @pl.when(ci < cpt - 1)
                    def _(k=k, d=d):
                        issue_feat(ch + 1, k + _D - nchunk, d)
                    issue_idx(k + _D - nchunk, d)

            for h in range(2):
                def wb(i, cc2, h=h):
                    offs = [h * _HALF + (i * 4 + u) * _L for u in range(4)]
                    accs = [acc_v[pl.ds(off, _L)] for off in offs]
                    rs = [recip_v[pl.ds(off, _L)] for off in offs]
                    for off, a, r in zip(offs, accs, rs):
                        acc_v[pl.ds(off, _L)] = a * r
                    return cc2

                lax.fori_loop(0, _HALF // (4 * _L), wb, 0)
                pltpu.async_copy(
                    acc_v.at[pl.ds(h * _HALF, _HALF)],
                    out_hbm.at[b, ch, pl.ds(h * _HALF, _HALF)],
                    osems[h])
            return cc

        lax.fori_loop(0, cpt, chan_body, 0)

        for h in range(2):
            pltpu.make_async_copy(
                acc_v.at[pl.ds(h * _HALF, _HALF)],
                out_hbm.at[b, ch0, pl.ds(h * _HALF, _HALF)],
                osems[h]).wait()
        for d in range(_D):
            pltpu.make_async_copy(
                idx_hbm.at[b, 0, pl.ds(0, _CH)], ibufs[d], isems[d]).wait()

    return main_k


def kernel(features, coords):
    B, C, N = features.shape
    nc, idx_pack = _normalize(coords)
    avg = _make_main_kernel(B, C, N)(features, idx_pack)
    return avg.reshape(B, C, _RES, _RES, _RES), nc
